# Initial kernel scaffold; baseline (speedup 1.0000x reference)
import jax
import jax.numpy as jnp
from jax.experimental import pallas as pl


def kernel(node_feature, graph_indicator):
    raise NotImplementedError("write your pallas kernel here")


# speedup vs baseline: 4.0558x; 4.0558x over previous
import jax
import jax.numpy as jnp
from jax import lax
from jax.experimental import pallas as pl
from jax.experimental.pallas import tpu as pltpu
from jax.experimental.pallas import tpu_sc as plsc

N = 100000
D = 128
S = 1024
CHUNK = 80
NCHUNKS = N // CHUNK
NWORKERS = 32
NJ_LO = NCHUNKS // NWORKERS
NJ_EXTRA = NCHUNKS % NWORKERS
RPS = S // 16


def _phase1_body(feat, ids, zsum_hbm, zcnt_hbm, ones_hbm, psum, pcnt,
                 fbuf, ibuf, ones_v, ssum, scnt):
    c = lax.axis_index("c")
    s = lax.axis_index("s")
    w = s * 2 + c

    nj = jnp.where(w < NJ_EXTRA, NJ_LO + 1, NJ_LO)

    pltpu.sync_copy(zsum_hbm.at[pl.ds(s * RPS, RPS)], ssum.at[pl.ds(s * RPS, RPS)])
    pltpu.sync_copy(zcnt_hbm.at[pl.ds(s * RPS, RPS)], scnt.at[pl.ds(s * RPS, RPS)])
    pltpu.sync_copy(ones_hbm, ones_v)
    plsc.subcore_barrier()

    def body_j(j, carry):
        r0 = (w + j * NWORKERS) * CHUNK
        pltpu.sync_copy(feat.at[pl.ds(r0, CHUNK)], fbuf)
        pltpu.sync_copy(ids.at[pl.ds(r0, CHUNK)], ibuf)
        pltpu.sync_copy(fbuf, ssum.at[ibuf], add=True)
        pltpu.sync_copy(ones_v, scnt.at[ibuf], add=True)
        return carry

    lax.fori_loop(0, nj, body_j, 0)
    plsc.subcore_barrier()

    pltpu.sync_copy(ssum.at[pl.ds(s * RPS, RPS)], psum.at[c, pl.ds(s * RPS, RPS)])
    pltpu.sync_copy(scnt.at[pl.ds(s * RPS, RPS)], pcnt.at[c, pl.ds(s * RPS, RPS)])


_phase1 = pl.kernel(
    _phase1_body,
    out_type=(
        jax.ShapeDtypeStruct((2, S, D), jnp.float32),
        jax.ShapeDtypeStruct((2, S, D), jnp.float32),
    ),
    mesh=plsc.VectorSubcoreMesh(core_axis_name="c", subcore_axis_name="s"),
    scratch_types=[
        pltpu.VMEM((CHUNK, D), jnp.float32),
        pltpu.VMEM((CHUNK,), jnp.int32),
        pltpu.VMEM((CHUNK, D), jnp.float32),
        pltpu.VMEM_SHARED((S, D), jnp.float32),
        pltpu.VMEM_SHARED((S, D), jnp.float32),
    ],
)


def _combine_body(psum_ref, pcnt_ref, out_ref):
    total = psum_ref[0] + psum_ref[1]
    cnt = pcnt_ref[0, :, 0:1] + pcnt_ref[1, :, 0:1]
    out_ref[...] = total / jnp.maximum(cnt, 1.0)


def kernel(node_feature, graph_indicator):
    ids = graph_indicator.astype(jnp.int32)
    zsum = jnp.zeros((S, D), jnp.float32)
    zcnt = jnp.zeros((S, D), jnp.float32)
    ones = jnp.ones((CHUNK, D), jnp.float32)
    psum, pcnt = _phase1(node_feature, ids, zsum, zcnt, ones)
    return pl.pallas_call(
        _combine_body,
        out_shape=jax.ShapeDtypeStruct((S, D), jnp.float32),
    )(psum, pcnt)


# speedup vs baseline: 6.5861x; 1.6239x over previous
import jax
import jax.numpy as jnp
from jax import lax
from jax.experimental import pallas as pl
from jax.experimental.pallas import tpu as pltpu
from jax.experimental.pallas import tpu_sc as plsc

N = 100000
D = 128
S = 1024
CHUNK = 80
NCHUNKS = N // CHUNK
NWORKERS = 32
NJ_LO = NCHUNKS // NWORKERS
NJ_EXTRA = NCHUNKS % NWORKERS
RPS = S // 16


def _phase1_body(feat, ids, zsum_hbm, zcnt_hbm, ones_hbm, psum, pcnt,
                 fbuf0, fbuf1, ibuf0, ibuf1, ones_v, ssum, scnt,
                 semf0, semf1, semi0, semi1):
    c = lax.axis_index("c")
    s = lax.axis_index("s")
    w = s * 2 + c

    nj = jnp.where(w < NJ_EXTRA, NJ_LO + 1, NJ_LO)

    pltpu.sync_copy(zsum_hbm.at[pl.ds(s * RPS, RPS)], ssum.at[pl.ds(s * RPS, RPS)])
    pltpu.sync_copy(zcnt_hbm.at[pl.ds(s * RPS, RPS)], scnt.at[pl.ds(s * RPS, RPS)])
    pltpu.sync_copy(ones_hbm, ones_v)
    plsc.subcore_barrier()

    def start_load(j, fbuf, ibuf, semf, semi):
        r0 = (w + j * NWORKERS) * CHUNK
        pltpu.make_async_copy(feat.at[pl.ds(r0, CHUNK)], fbuf, semf).start()
        pltpu.make_async_copy(ids.at[pl.ds(r0, CHUNK)], ibuf, semi).start()

    def finish_and_reduce(j, fbuf, ibuf, semf, semi):
        r0 = (w + j * NWORKERS) * CHUNK
        pltpu.make_async_copy(feat.at[pl.ds(r0, CHUNK)], fbuf, semf).wait()
        pltpu.make_async_copy(ids.at[pl.ds(r0, CHUNK)], ibuf, semi).wait()
        pltpu.sync_copy(fbuf, ssum.at[ibuf], add=True)
        pltpu.sync_copy(ones_v, scnt.at[ibuf], add=True)

    start_load(0, fbuf0, ibuf0, semf0, semi0)

    def body_j(j, carry):
        @pl.when(j % 2 == 0)
        def _():
            @pl.when(j + 1 < nj)
            def _():
                start_load(j + 1, fbuf1, ibuf1, semf1, semi1)
            finish_and_reduce(j, fbuf0, ibuf0, semf0, semi0)

        @pl.when(j % 2 == 1)
        def _():
            @pl.when(j + 1 < nj)
            def _():
                start_load(j + 1, fbuf0, ibuf0, semf0, semi0)
            finish_and_reduce(j, fbuf1, ibuf1, semf1, semi1)

        return carry

    lax.fori_loop(0, nj, body_j, 0)
    plsc.subcore_barrier()

    pltpu.sync_copy(ssum.at[pl.ds(s * RPS, RPS)], psum.at[c, pl.ds(s * RPS, RPS)])
    pltpu.sync_copy(scnt.at[pl.ds(s * RPS, RPS)], pcnt.at[c, pl.ds(s * RPS, RPS)])


_phase1 = pl.kernel(
    _phase1_body,
    out_type=(
        jax.ShapeDtypeStruct((2, S, D), jnp.float32),
        jax.ShapeDtypeStruct((2, S, D), jnp.float32),
    ),
    mesh=plsc.VectorSubcoreMesh(core_axis_name="c", subcore_axis_name="s"),
    scratch_types=[
        pltpu.VMEM((CHUNK, D), jnp.float32),
        pltpu.VMEM((CHUNK, D), jnp.float32),
        pltpu.VMEM((CHUNK,), jnp.int32),
        pltpu.VMEM((CHUNK,), jnp.int32),
        pltpu.VMEM((CHUNK, D), jnp.float32),
        pltpu.VMEM_SHARED((S, D), jnp.float32),
        pltpu.VMEM_SHARED((S, D), jnp.float32),
        pltpu.SemaphoreType.DMA,
        pltpu.SemaphoreType.DMA,
        pltpu.SemaphoreType.DMA,
        pltpu.SemaphoreType.DMA,
    ],
)


def _combine_body(psum_ref, pcnt_ref, out_ref):
    total = psum_ref[0] + psum_ref[1]
    cnt = pcnt_ref[0, :, 0:1] + pcnt_ref[1, :, 0:1]
    out_ref[...] = total / jnp.maximum(cnt, 1.0)


def kernel(node_feature, graph_indicator):
    ids = graph_indicator.astype(jnp.int32)
    zsum = jnp.zeros((S, D), jnp.float32)
    zcnt = jnp.zeros((S, D), jnp.float32)
    ones = jnp.ones((CHUNK, D), jnp.float32)
    psum, pcnt = _phase1(node_feature, ids, zsum, zcnt, ones)
    return pl.pallas_call(
        _combine_body,
        out_shape=jax.ShapeDtypeStruct((S, D), jnp.float32),
    )(psum, pcnt)


# speedup vs baseline: 6.7252x; 1.0211x over previous
import jax
import jax.numpy as jnp
from jax import lax
from jax.experimental import pallas as pl
from jax.experimental.pallas import tpu as pltpu
from jax.experimental.pallas import tpu_sc as plsc

N = 100000
D = 128
S = 1024
CHUNK = 80
MERGE = 2
MCHUNK = CHUNK * MERGE
NCHUNKS = N // MCHUNK
NWORKERS = 32
NJ_LO = NCHUNKS // NWORKERS
NJ_EXTRA = NCHUNKS % NWORKERS
RPS = S // 16


def _phase1_body(feat, ids, zsum_hbm, zcnt_hbm, ones_hbm, psum, pcnt,
                 fbuf0, fbuf1, ibufa0, ibufb0, ibufa1, ibufb1, ones_v,
                 ssum, scnt,
                 semf0, semf1, semia0, semib0, semia1, semib1):
    c = lax.axis_index("c")
    s = lax.axis_index("s")
    w = s * 2 + c

    nj = jnp.where(w < NJ_EXTRA, NJ_LO + 1, NJ_LO)

    pltpu.sync_copy(zsum_hbm.at[pl.ds(s * RPS, RPS)], ssum.at[pl.ds(s * RPS, RPS)])
    pltpu.sync_copy(zcnt_hbm.at[pl.ds(s * RPS, RPS)], scnt.at[pl.ds(s * RPS, RPS)])
    pltpu.sync_copy(ones_hbm, ones_v)
    plsc.subcore_barrier()

    def start_load(j, fbuf, ibufa, ibufb, semf, semia, semib):
        r0 = (w + j * NWORKERS) * MCHUNK
        pltpu.make_async_copy(feat.at[pl.ds(r0, MCHUNK)], fbuf, semf).start()
        pltpu.make_async_copy(ids.at[pl.ds(r0, CHUNK)], ibufa, semia).start()
        pltpu.make_async_copy(ids.at[pl.ds(r0 + CHUNK, CHUNK)], ibufb, semib).start()

    def finish_and_reduce(j, fbuf, ibufa, ibufb, semf, semia, semib):
        r0 = (w + j * NWORKERS) * MCHUNK
        pltpu.make_async_copy(feat.at[pl.ds(r0, MCHUNK)], fbuf, semf).wait()
        pltpu.make_async_copy(ids.at[pl.ds(r0, CHUNK)], ibufa, semia).wait()
        pltpu.make_async_copy(ids.at[pl.ds(r0 + CHUNK, CHUNK)], ibufb, semib).wait()
        pltpu.sync_copy(fbuf.at[pl.ds(0, CHUNK)], ssum.at[ibufa], add=True)
        pltpu.sync_copy(ones_v, scnt.at[ibufa], add=True)
        pltpu.sync_copy(fbuf.at[pl.ds(CHUNK, CHUNK)], ssum.at[ibufb], add=True)
        pltpu.sync_copy(ones_v, scnt.at[ibufb], add=True)

    start_load(0, fbuf0, ibufa0, ibufb0, semf0, semia0, semib0)

    def body_j(j, carry):
        @pl.when(j % 2 == 0)
        def _():
            @pl.when(j + 1 < nj)
            def _():
                start_load(j + 1, fbuf1, ibufa1, ibufb1, semf1, semia1, semib1)
            finish_and_reduce(j, fbuf0, ibufa0, ibufb0, semf0, semia0, semib0)

        @pl.when(j % 2 == 1)
        def _():
            @pl.when(j + 1 < nj)
            def _():
                start_load(j + 1, fbuf0, ibufa0, ibufb0, semf0, semia0, semib0)
            finish_and_reduce(j, fbuf1, ibufa1, ibufb1, semf1, semia1, semib1)

        return carry

    lax.fori_loop(0, nj, body_j, 0)
    plsc.subcore_barrier()

    pltpu.sync_copy(ssum.at[pl.ds(s * RPS, RPS)], psum.at[c, pl.ds(s * RPS, RPS)])
    pltpu.sync_copy(scnt.at[pl.ds(s * RPS, RPS)], pcnt.at[c, pl.ds(s * RPS, RPS)])


_phase1 = pl.kernel(
    _phase1_body,
    out_type=(
        jax.ShapeDtypeStruct((2, S, D), jnp.float32),
        jax.ShapeDtypeStruct((2, S, D), jnp.float32),
    ),
    mesh=plsc.VectorSubcoreMesh(core_axis_name="c", subcore_axis_name="s"),
    scratch_types=[
        pltpu.VMEM((MCHUNK, D), jnp.float32),
        pltpu.VMEM((MCHUNK, D), jnp.float32),
        pltpu.VMEM((CHUNK,), jnp.int32),
        pltpu.VMEM((CHUNK,), jnp.int32),
        pltpu.VMEM((CHUNK,), jnp.int32),
        pltpu.VMEM((CHUNK,), jnp.int32),
        pltpu.VMEM((CHUNK, D), jnp.float32),
        pltpu.VMEM_SHARED((S, D), jnp.float32),
        pltpu.VMEM_SHARED((S, D), jnp.float32),
        pltpu.SemaphoreType.DMA,
        pltpu.SemaphoreType.DMA,
        pltpu.SemaphoreType.DMA,
        pltpu.SemaphoreType.DMA,
        pltpu.SemaphoreType.DMA,
        pltpu.SemaphoreType.DMA,
    ],
)


def _combine_body(psum_ref, pcnt_ref, out_ref):
    total = psum_ref[0] + psum_ref[1]
    cnt = pcnt_ref[0, :, 0:1] + pcnt_ref[1, :, 0:1]
    out_ref[...] = total / jnp.maximum(cnt, 1.0)


def kernel(node_feature, graph_indicator):
    ids = graph_indicator.astype(jnp.int32)
    zsum = jnp.zeros((S, D), jnp.float32)
    zcnt = jnp.zeros((S, D), jnp.float32)
    ones = jnp.ones((CHUNK, D), jnp.float32)
    psum, pcnt = _phase1(node_feature, ids, zsum, zcnt, ones)
    return pl.pallas_call(
        _combine_body,
        out_shape=jax.ShapeDtypeStruct((S, D), jnp.float32),
    )(psum, pcnt)
